# baseline (speedup 1.0000x reference)
import jax
import jax.numpy as jnp
from jax.experimental import pallas as pl
from jax.experimental.pallas import tpu as pltpu

B = 512


def _leaky_relu(x, s):
    return jnp.where(x >= 0, x, s * x)


def _elu(x):
    return jnp.where(x > 0, x, jnp.expm1(x))


def _seg_softmax(a, seg, n):
    m = jax.ops.segment_max(a, seg, num_segments=n)
    m = jnp.where(jnp.isfinite(m), m, 0.0)
    e = jnp.exp(a - m[seg])
    s = jax.ops.segment_sum(e, seg, num_segments=n)
    return e / (s[seg] + 1e-16)


def _gru(inp, h, Wi, Wh, bi, bh):
    gi = inp @ Wi + bi
    gh = h @ Wh + bh
    i_r, i_z, i_n = jnp.split(gi, 3, axis=-1)
    h_r, h_z, h_n = jnp.split(gh, 3, axis=-1)
    r = jax.nn.sigmoid(i_r + h_r)
    z = jax.nn.sigmoid(i_z + h_z)
    ng = jnp.tanh(i_n + r * h_n)
    return (1.0 - z) * ng + z * h


def _lin2_kernel(x_ref, w_ref, b_ref, o_ref):
    o_ref[...] = jnp.dot(x_ref[...], w_ref[...],
                         preferred_element_type=jnp.float32) + b_ref[...]


def _lin2(x, W, b):
    return pl.pallas_call(
        _lin2_kernel,
        out_shape=jax.ShapeDtypeStruct((x.shape[0], W.shape[1]), jnp.float32),
    )(x, W, b[None, :])


def kernel(x, edge_index, edge_attr, batch, lin1_W, lin1_b, c0_W, c0_asrc, c0_adst, c0_b, c0_We, c0_aedge, g0_Wi, g0_Wh, g0_bi, g0_bh, c1_W, c1_asrc, c1_adst, c1_b, g1_Wi, g1_Wh, g1_bi, g1_bh, cm_W, cm_asrc, cm_adst, cm_b, gm_Wi, gm_Wh, gm_bi, gm_bh, lin2_W, lin2_b):
    src, dst = edge_index[0], edge_index[1]
    n = x.shape[0]
    x0 = _leaky_relu(x @ lin1_W + lin1_b, 0.01)
    ones_e = jnp.ones((src.shape[0],), jnp.float32)
    cnt = jax.ops.segment_sum(ones_e, dst, num_segments=n)
    mean_attr = jax.ops.segment_sum(edge_attr, dst, num_segments=n) / jnp.maximum(cnt, 1.0)[:, None]
    loop = jnp.arange(n)
    srcf = jnp.concatenate([src, loop])
    dstf = jnp.concatenate([dst, loop])
    ea = jnp.concatenate([edge_attr, mean_attr], axis=0)
    hs = x0 @ c0_W
    ef = ea @ c0_We
    al = (hs * c0_asrc).sum(-1)[srcf] + (hs * c0_adst).sum(-1)[dstf] + (ef * c0_aedge).sum(-1)
    al = _seg_softmax(_leaky_relu(al, 0.2), dstf, n)
    h = jax.ops.segment_sum(hs[srcf] * al[:, None], dstf, num_segments=n) + c0_b
    h = _elu(h)
    x1 = jax.nn.relu(_gru(h, x0, g0_Wi, g0_Wh, g0_bi, g0_bh))
    hs = x1 @ c1_W
    al = (hs * c1_asrc).sum(-1)[src] + (hs * c1_adst).sum(-1)[dst]
    al = _seg_softmax(_leaky_relu(al, 0.01), dst, n)
    h = jax.ops.segment_sum(hs[src] * al[:, None], dst, num_segments=n) + c1_b
    h = _elu(h)
    x2 = jax.nn.relu(_gru(h, x1, g1_Wi, g1_Wh, g1_bi, g1_bh))
    out = jax.nn.relu(jax.ops.segment_sum(x2, batch, num_segments=B))
    for _ in range(2):
        hs_m = x2 @ cm_W
        hd_m = out @ cm_W
        al = (hs_m * cm_asrc).sum(-1) + (hd_m * cm_adst).sum(-1)[batch]
        al = _seg_softmax(_leaky_relu(al, 0.01), batch, B)
        hm = jax.ops.segment_sum(hs_m * al[:, None], batch, num_segments=B) + cm_b
        hm = _elu(hm)
        out = jax.nn.relu(_gru(hm, out, gm_Wi, gm_Wh, gm_bi, gm_bh))
    return _lin2(out, lin2_W, lin2_b)


# speedup vs baseline: 11.3419x; 11.3419x over previous
import functools

import jax
import jax.numpy as jnp
from jax import lax
from jax.experimental import pallas as pl
from jax.experimental.pallas import tpu as pltpu
from jax.experimental.pallas import tpu_sc as plsc

N = 10000
E = 320000
D = 128
HID = 128
ED = 16
B = 512
OUT = 128

NC = 2
NS = 16
NW = NC * NS
EPW = E // NW
C = 80
NCH = EPW // C
ZR = 16
MINI = 16
GRP = C // 16

_SC_PARAMS = pltpu.CompilerParams(needs_layout_passes=False)


def _mesh():
    return plsc.VectorSubcoreMesh(
        core_axis_name="c", subcore_axis_name="s",
        num_cores=NC, num_subcores=NS)


def _zero_rows(ref, nrows, width):
    def body(i, _):
        for k in range(width // 16):
            ref[i, pl.ds(k * 16, 16)] = jnp.zeros((16,), jnp.float32)
        return 0
    lax.fori_loop(0, nrows, body, 0)


def _zero_shared(zsrc, sh, sid):
    def z(j, _):
        r0 = pl.multiple_of(j * (16 * NS) + sid * ZR, 8)
        pltpu.sync_copy(zsrc, sh.at[pl.ds(r0, ZR)])
        return 0
    lax.fori_loop(0, N // (16 * NS), z, 0)

    @pl.when(sid == 0)
    def _():
        pltpu.sync_copy(zsrc, sh.at[pl.ds((N // (16 * NS)) * (16 * NS), ZR)])


def _copy_out(sh, out, cid, sid):
    def cp(j, _):
        r0 = pl.multiple_of(j * (16 * NS) + sid * ZR, 8)
        pltpu.sync_copy(sh.at[pl.ds(r0, ZR)], out.at[cid, pl.ds(r0, ZR)])
        return 0
    lax.fori_loop(0, N // (16 * NS), cp, 0)

    @pl.when(sid == 0)
    def _():
        r0 = (N // (16 * NS)) * (16 * NS)
        pltpu.sync_copy(sh.at[pl.ds(r0, ZR)], out.at[cid, pl.ds(r0, ZR)])


def _scal_body(slope, has_pe, *refs):
    it = iter(refs)
    src_hbm = next(it)
    dst_hbm = next(it)
    pe_hbm = next(it) if has_pe else None
    as_hbm = next(it)
    ad_hbm = next(it)
    gb_hbm = next(it)
    e_out = next(it)
    sp_out = next(it)
    if has_pe:
        cp_out = next(it)
        wp_out = next(it)
    asv = next(it)
    adv = next(it)
    src_v = next(it)
    dst_v = next(it)
    pe_v = next(it)
    e_v = next(it)
    gb_v = next(it)
    sa_acc = next(it)
    if has_pe:
        ca_acc = next(it)
        wa_acc = next(it)

    cid = lax.axis_index("c")
    sid = lax.axis_index("s")
    wid = sid * NC + cid

    pltpu.sync_copy(as_hbm, asv)
    pltpu.sync_copy(ad_hbm, adv)
    pltpu.sync_copy(gb_hbm, gb_v)
    gb = gb_v[...]

    zeros_f = jnp.zeros((16,), jnp.float32)

    def zacc(i, _):
        sl = pl.ds(i * 16, 16)
        sa_acc[sl] = zeros_f
        if has_pe:
            ca_acc[sl] = zeros_f
            wa_acc[sl] = zeros_f
        return 0
    lax.fori_loop(0, N // 16, zacc, 0)

    ones_f = jnp.ones((16,), jnp.float32)

    def chunk(i, _):
        base = wid * EPW + i * C
        pltpu.sync_copy(src_hbm.at[pl.ds(base, C)], src_v)
        pltpu.sync_copy(dst_hbm.at[pl.ds(base, C)], dst_v)
        if has_pe:
            pltpu.sync_copy(pe_hbm.at[pl.ds(base, C)], pe_v)
        for g in range(GRP):
            sl = pl.ds(g * 16, 16)
            si = src_v[sl]
            di = dst_v[sl]
            a = plsc.load_gather(asv, [si]) + plsc.load_gather(adv, [di])
            if has_pe:
                pe = pe_v[sl]
                a = a + pe
            a = jnp.where(a >= 0.0, a, slope * a)
            e = jnp.exp(a - gb)
            e_v[sl] = e
            plsc.addupdate_scatter(sa_acc, [di], e)
            if has_pe:
                plsc.addupdate_scatter(ca_acc, [di], ones_f)
                plsc.addupdate_scatter(wa_acc, [di], pe)
        pltpu.sync_copy(e_v, e_out.at[pl.ds(base, C)])
        return 0

    lax.fori_loop(0, NCH, chunk, 0)
    pltpu.sync_copy(sa_acc, sp_out.at[pl.ds(wid * N, N)])
    if has_pe:
        pltpu.sync_copy(ca_acc, cp_out.at[pl.ds(wid * N, N)])
        pltpu.sync_copy(wa_acc, wp_out.at[pl.ds(wid * N, N)])


def _rows_body(*refs):
    it = iter(refs)
    hs_hbm = next(it)
    src_hbm = next(it)
    dst_hbm = next(it)
    e_hbm = next(it)
    u_out = next(it)
    src_v = next(it)
    dst_v = next(it)
    e_v = next(it)
    rows_v = next(it)
    u_sh = next(it)
    gsem = next(it)

    cid = lax.axis_index("c")
    sid = lax.axis_index("s")
    wid = sid * NC + cid

    _zero_rows(rows_v, ZR, D)
    _zero_shared(rows_v.at[pl.ds(0, ZR)], u_sh, sid)
    plsc.subcore_barrier()

    def chunk(i, _):
        base = wid * EPW + i * C
        pltpu.sync_copy(src_hbm.at[pl.ds(base, C)], src_v)
        pltpu.sync_copy(dst_hbm.at[pl.ds(base, C)], dst_v)
        pltpu.sync_copy(e_hbm.at[pl.ds(base, C)], e_v)
        pltpu.async_copy(hs_hbm.at[src_v], rows_v, gsem).wait()
        for g in range(GRP):
            eg = e_v[pl.ds(g * 16, 16)]
            for l in range(16):
                j = g * 16 + l
                ej = eg[l]
                for k in range(D // 16):
                    sl = pl.ds(k * 16, 16)
                    rows_v[j, sl] = rows_v[j, sl] * ej
        pltpu.sync_copy(rows_v, u_sh.at[dst_v], add=True)
        return 0

    lax.fori_loop(0, NCH, chunk, 0)
    plsc.subcore_barrier()
    _copy_out(u_sh, u_out, cid, sid)


def _sc_scalar(src, dst, pe, asn, adn, gb, slope):
    has_pe = pe is not None
    scratch = [
        pltpu.VMEM((N,), jnp.float32),
        pltpu.VMEM((N,), jnp.float32),
        pltpu.VMEM((C,), jnp.int32),
        pltpu.VMEM((C,), jnp.int32),
        pltpu.VMEM((C,), jnp.float32),
        pltpu.VMEM((C,), jnp.float32),
        pltpu.VMEM((16,), jnp.float32),
        pltpu.VMEM((N,), jnp.float32),
    ]
    out_type = [
        jax.ShapeDtypeStruct((E,), jnp.float32),
        jax.ShapeDtypeStruct((NW * N,), jnp.float32),
    ]
    if has_pe:
        scratch += [
            pltpu.VMEM((N,), jnp.float32),
            pltpu.VMEM((N,), jnp.float32),
        ]
        out_type += [
            jax.ShapeDtypeStruct((NW * N,), jnp.float32),
            jax.ShapeDtypeStruct((NW * N,), jnp.float32),
        ]
    body = functools.partial(_scal_body, slope, has_pe)
    fn = pl.kernel(body, out_type=out_type, mesh=_mesh(),
                   scratch_types=scratch, compiler_params=_SC_PARAMS)
    args = (src, dst, pe, asn, adn, gb) if has_pe else (src, dst, asn, adn, gb)
    return fn(*args)


def _sc_rows(hs, src, dst, e):
    scratch = [
        pltpu.VMEM((C,), jnp.int32),
        pltpu.VMEM((C,), jnp.int32),
        pltpu.VMEM((C,), jnp.float32),
        pltpu.VMEM((C, D), jnp.float32),
        pltpu.VMEM_SHARED((N, D), jnp.float32),
        pltpu.SemaphoreType.DMA,
    ]
    out_type = jax.ShapeDtypeStruct((NC, N, D), jnp.float32)
    fn = pl.kernel(_rows_body, out_type=out_type, mesh=_mesh(),
                   scratch_types=scratch, compiler_params=_SC_PARAMS)
    return fn(hs, src, dst, e)


def _leaky_relu(x, s):
    return jnp.where(x >= 0, x, s * x)


def _elu(x):
    return jnp.where(x > 0, x, jnp.expm1(x))


def _seg_softmax(a, seg, n):
    m = jax.ops.segment_max(a, seg, num_segments=n)
    m = jnp.where(jnp.isfinite(m), m, 0.0)
    e = jnp.exp(a - m[seg])
    s = jax.ops.segment_sum(e, seg, num_segments=n)
    return e / (s[seg] + 1e-16)


def _gru(inp, h, Wi, Wh, bi, bh):
    gi = inp @ Wi + bi
    gh = h @ Wh + bh
    i_r, i_z, i_n = jnp.split(gi, 3, axis=-1)
    h_r, h_z, h_n = jnp.split(gh, 3, axis=-1)
    r = jax.nn.sigmoid(i_r + h_r)
    z = jax.nn.sigmoid(i_z + h_z)
    ng = jnp.tanh(i_n + r * h_n)
    return (1.0 - z) * ng + z * h


def _lin2_kernel(x_ref, w_ref, b_ref, o_ref):
    o_ref[...] = jnp.dot(x_ref[...], w_ref[...],
                         preferred_element_type=jnp.float32) + b_ref[...]


def _lin2(x, W, b):
    return pl.pallas_call(
        _lin2_kernel,
        out_shape=jax.ShapeDtypeStruct((x.shape[0], W.shape[1]), jnp.float32),
    )(x, W, b[None, :])


def _conv_sc(hs, src, dst, pe, asn, adn, gb, slope):
    gbv = jnp.full((16,), 1.0, jnp.float32) * gb
    if pe is not None:
        e, sp, cp, wp = _sc_scalar(src, dst, pe, asn, adn, gbv, slope)
        cnt = cp.reshape(NW, N).sum(0)
        sew = wp.reshape(NW, N).sum(0)
    else:
        e, sp = _sc_scalar(src, dst, None, asn, adn, gbv, slope)
        cnt = sew = None
    s = sp.reshape(NW, N).sum(0)
    U = _sc_rows(hs, src, dst, e)
    return U[0] + U[1], s, cnt, sew


def kernel(x, edge_index, edge_attr, batch, lin1_W, lin1_b, c0_W, c0_asrc, c0_adst, c0_b, c0_We, c0_aedge, g0_Wi, g0_Wh, g0_bi, g0_bh, c1_W, c1_asrc, c1_adst, c1_b, g1_Wi, g1_Wh, g1_bi, g1_bh, cm_W, cm_asrc, cm_adst, cm_b, gm_Wi, gm_Wh, gm_bi, gm_bh, lin2_W, lin2_b):
    src, dst = edge_index[0], edge_index[1]
    x0 = _leaky_relu(x @ lin1_W + lin1_b, 0.01)

    hs0 = x0 @ c0_W
    as0 = hs0 @ c0_asrc
    ad0 = hs0 @ c0_adst
    we = c0_We @ c0_aedge
    ew = edge_attr @ we
    gb0 = jnp.maximum(as0.max() + ad0.max() + jnp.maximum(ew.max(), 0.0), 0.0)
    Uc, s, cnt, sew = _conv_sc(hs0, src, dst, ew, as0, ad0, gb0, 0.2)
    mean_ew = sew / jnp.maximum(cnt, 1.0)
    e_self = jnp.exp(_leaky_relu(as0 + ad0 + mean_ew, 0.2) - gb0)
    Uc = Uc + e_self[:, None] * hs0
    s = s + e_self
    h = _elu(Uc / (s + 1e-16)[:, None] + c0_b)
    x1 = jax.nn.relu(_gru(h, x0, g0_Wi, g0_Wh, g0_bi, g0_bh))

    hs1 = x1 @ c1_W
    as1 = hs1 @ c1_asrc
    ad1 = hs1 @ c1_adst
    gb1 = jnp.maximum(as1.max() + ad1.max(), 0.0)
    U1, s1, _, _ = _conv_sc(hs1, src, dst, None, as1, ad1, gb1, 0.01)
    h1 = _elu(U1 / (s1 + 1e-16)[:, None] + c1_b)
    x2 = jax.nn.relu(_gru(h1, x1, g1_Wi, g1_Wh, g1_bi, g1_bh))

    out = jax.nn.relu(jax.ops.segment_sum(x2, batch, num_segments=B))
    for _ in range(2):
        hs_m = x2 @ cm_W
        hd_m = out @ cm_W
        al = (hs_m * cm_asrc).sum(-1) + (hd_m * cm_adst).sum(-1)[batch]
        al = _seg_softmax(_leaky_relu(al, 0.01), batch, B)
        hm = jax.ops.segment_sum(hs_m * al[:, None], batch, num_segments=B) + cm_b
        hm = _elu(hm)
        out = jax.nn.relu(_gru(hm, out, gm_Wi, gm_Wh, gm_bi, gm_bh))
    return _lin2(out, lin2_W, lin2_b)
